# Initial kernel scaffold; baseline (speedup 1.0000x reference)
import jax
import jax.numpy as jnp
from jax.experimental import pallas as pl


def kernel(enc_input, Wq, Wk, Wv, Wfc, ln_attn_g, ln_attn_b, Wg1, W1, b1, W2, b2, ln1_g, ln1_b, ln2_g, ln2_b, ln3_g, ln3_b, ln4_g, ln4_b, lnf_g, lnf_b):
    raise NotImplementedError("write your pallas kernel here")


# speedup vs baseline: 1.7646x; 1.7646x over previous
import functools

import jax
import jax.numpy as jnp
from jax import lax
from jax.experimental import pallas as pl
from jax.experimental.pallas import tpu as pltpu
from jax.experimental.pallas import tpu_sc as plsc

N, D, DI = 2048, 768, 1536
H, DK, DV = 12, 64, 64
E, TOPK = 24, 2
P = N * TOPK
T = 128
NT = P // T + E
PPAD = NT * T
EPS = 1e-6

NWORK = 32


def _qkv_body(x_ref, w_ref, o_ref):
    o_ref[...] = jnp.dot(x_ref[...], w_ref[...],
                         preferred_element_type=jnp.float32)


def _qkv(x, wqkv):
    return pl.pallas_call(
        _qkv_body,
        grid=(8,),
        in_specs=[pl.BlockSpec((N // 8, D), lambda i: (i, 0)),
                  pl.BlockSpec((D, 3 * D), lambda i: (0, 0))],
        out_specs=pl.BlockSpec((N // 8, 3 * D), lambda i: (i, 0)),
        out_shape=jax.ShapeDtypeStruct((N, 3 * D), jnp.float32),
    )(x, wqkv)


def _attn_body(q_ref, k_ref, v_ref, o_ref):
    q = q_ref[0] * (1.0 / (DK ** 0.5))
    s = lax.dot_general(q, k_ref[0], (((1,), (1,)), ((), ())),
                        preferred_element_type=jnp.float32)
    m = jnp.max(s, axis=-1, keepdims=True)
    p = jnp.exp(s - m)
    p = p / jnp.sum(p, axis=-1, keepdims=True)
    o_ref[0] = jnp.dot(p, v_ref[0], preferred_element_type=jnp.float32)


def _attn(qkvh):
    qb = 512
    return pl.pallas_call(
        _attn_body,
        grid=(H, N // qb),
        in_specs=[pl.BlockSpec((1, qb, DK), lambda h, i: (h, i, 0)),
                  pl.BlockSpec((1, N, DK), lambda h, i: (H + h, 0, 0)),
                  pl.BlockSpec((1, N, DV), lambda h, i: (2 * H + h, 0, 0))],
        out_specs=pl.BlockSpec((1, qb, DV), lambda h, i: (h, i, 0)),
        out_shape=jax.ShapeDtypeStruct((H, N, DV), jnp.float32),
    )(qkvh, qkvh, qkvh)


def _post_attn_body(ctx_ref, res_ref, wfc_ref, g_ref, b_ref, wg_ref,
                    x_ref, e1_ref, e2_ref, w1_ref, w2_ref):
    c = jnp.dot(ctx_ref[...], wfc_ref[...], preferred_element_type=jnp.float32)
    h = jnp.maximum(c, 0.0) + res_ref[...]
    m = jnp.mean(h, axis=-1, keepdims=True)
    v = jnp.mean((h - m) ** 2, axis=-1, keepdims=True)
    xln = (h - m) / jnp.sqrt(v + EPS) * g_ref[...] + b_ref[...]
    x_ref[...] = xln
    logits = jnp.dot(xln, wg_ref[...], preferred_element_type=jnp.float32)
    lm = jnp.max(logits, axis=-1, keepdims=True)
    pexp = jnp.exp(logits - lm)
    p = pexp / jnp.sum(pexp, axis=-1, keepdims=True)
    idx = lax.broadcasted_iota(jnp.int32, p.shape, 1)
    m1 = jnp.max(p, axis=-1, keepdims=True)
    e1 = jnp.min(jnp.where(p == m1, idx, E), axis=-1, keepdims=True)
    p2 = jnp.where(idx == e1, -1.0, p)
    m2 = jnp.max(p2, axis=-1, keepdims=True)
    e2 = jnp.min(jnp.where(p2 == m2, idx, E), axis=-1, keepdims=True)
    tot = m1 + m2
    e1_ref[...] = e1
    e2_ref[...] = e2
    w1_ref[...] = m1 / tot
    w2_ref[...] = m2 / tot


def _post_attn(ctx, res, wfc, g, b, wg):
    rb = 256
    return pl.pallas_call(
        _post_attn_body,
        grid=(N // rb,),
        in_specs=[pl.BlockSpec((rb, H * DV), lambda i: (i, 0)),
                  pl.BlockSpec((rb, D), lambda i: (i, 0)),
                  pl.BlockSpec((H * DV, D), lambda i: (0, 0)),
                  pl.BlockSpec((1, D), lambda i: (0, 0)),
                  pl.BlockSpec((1, D), lambda i: (0, 0)),
                  pl.BlockSpec((D, E), lambda i: (0, 0))],
        out_specs=[pl.BlockSpec((rb, D), lambda i: (i, 0)),
                   pl.BlockSpec((rb, 1), lambda i: (i, 0)),
                   pl.BlockSpec((rb, 1), lambda i: (i, 0)),
                   pl.BlockSpec((rb, 1), lambda i: (i, 0)),
                   pl.BlockSpec((rb, 1), lambda i: (i, 0))],
        out_shape=[jax.ShapeDtypeStruct((N, D), jnp.float32),
                   jax.ShapeDtypeStruct((N, 1), jnp.int32),
                   jax.ShapeDtypeStruct((N, 1), jnp.int32),
                   jax.ShapeDtypeStruct((N, 1), jnp.float32),
                   jax.ShapeDtypeStruct((N, 1), jnp.float32)],
    )(ctx, res, wfc, g.reshape(1, D), b.reshape(1, D), wg)


def _route_body(e_ref, dst_ref, te_ref):
    ep = e_ref[...]
    iota_e = lax.broadcasted_iota(jnp.int32, (P, E), 1)
    oh = (ep == iota_e).astype(jnp.float32)
    nc = P // T
    oh3 = oh.reshape(nc, T, E)
    r3 = lax.broadcasted_iota(jnp.int32, (T, T), 0)
    c3 = lax.broadcasted_iota(jnp.int32, (T, T), 1)
    tri_t = (c3 < r3).astype(jnp.float32)
    tri_b = jnp.broadcast_to(tri_t, (nc, T, T))
    within = lax.dot_general(tri_b, oh3, (((2,), (1,)), ((0,), (0,))),
                             preferred_element_type=jnp.float32)
    csum = jnp.sum(oh3, axis=1)
    rc = lax.broadcasted_iota(jnp.int32, (nc, nc), 0)
    cc = lax.broadcasted_iota(jnp.int32, (nc, nc), 1)
    tri_c = (cc < rc).astype(jnp.float32)
    carry = jnp.dot(tri_c, csum, preferred_element_type=jnp.float32)
    rank = (within + carry[:, None, :]).reshape(P, E)
    counts = jnp.sum(csum, axis=0, keepdims=True)
    padded = jnp.ceil(counts * (1.0 / T)) * T
    re = lax.broadcasted_iota(jnp.int32, (E, E), 0)
    ce = lax.broadcasted_iota(jnp.int32, (E, E), 1)
    triu_e = (re < ce).astype(jnp.float32)
    pad_off = jnp.dot(padded, triu_e,
                      preferred_element_type=jnp.float32)
    dst = jnp.sum((rank + pad_off) * oh, axis=-1, keepdims=True)
    dst_ref[...] = dst.astype(jnp.int32)
    ends = pad_off + padded
    ti = lax.broadcasted_iota(jnp.int32, (64, E), 0).astype(jnp.float32) * float(T)
    te = jnp.sum((ends <= ti).astype(jnp.float32), axis=-1, keepdims=True)
    te_ref[...] = jnp.minimum(te, float(E - 1)).astype(jnp.int32)


def _route(e_pairs):
    return pl.pallas_call(
        _route_body,
        grid=(1,),
        in_specs=[pl.BlockSpec((P, 1), lambda i: (0, 0))],
        out_specs=[pl.BlockSpec((P, 1), lambda i: (0, 0)),
                   pl.BlockSpec((64, 1), lambda i: (0, 0))],
        out_shape=[jax.ShapeDtypeStruct((P, 1), jnp.int32),
                   jax.ShapeDtypeStruct((64, 1), jnp.int32)],
    )(e_pairs)


def _expert_mm_body(te_ref, x_ref, w_ref, b_ref, o_ref):
    o_ref[...] = (jnp.dot(x_ref[...], w_ref[0],
                          preferred_element_type=jnp.float32) + b_ref[0])


def _expert_mm(te, xs, w, bias, din, dout):
    gs = pltpu.PrefetchScalarGridSpec(
        num_scalar_prefetch=1,
        grid=(NT,),
        in_specs=[pl.BlockSpec((T, din), lambda i, te: (i, 0)),
                  pl.BlockSpec((1, din, dout), lambda i, te: (te[i], 0, 0)),
                  pl.BlockSpec((1, 1, dout), lambda i, te: (te[i], 0, 0))],
        out_specs=pl.BlockSpec((T, dout), lambda i, te: (i, 0)),
    )
    return pl.pallas_call(
        _expert_mm_body,
        grid_spec=gs,
        out_shape=jax.ShapeDtypeStruct((PPAD, dout), jnp.float32),
    )(te, xs, w, bias.reshape(E, 1, dout))


def _mid_body(a_ref, b2_ref, w1_ref, w2_ref, g_ref, bb_ref, o_ref):
    y = w1_ref[...] * a_ref[...] + w2_ref[...] * b2_ref[...]
    m = jnp.mean(y, axis=-1, keepdims=True)
    v = jnp.mean((y - m) ** 2, axis=-1, keepdims=True)
    y = (y - m) / jnp.sqrt(v + EPS) * g_ref[0] + bb_ref[0]
    o_ref[...] = jnp.maximum(y, 0.0)


def _mid(a, b, w1, w2, lng, lnb):
    cb = N // 4
    return pl.pallas_call(
        _mid_body,
        grid=(4,),
        in_specs=[pl.BlockSpec((cb, DI), lambda i: (i, 0)),
                  pl.BlockSpec((cb, DI), lambda i: (i, 0)),
                  pl.BlockSpec((cb, 1), lambda i: (i, 0)),
                  pl.BlockSpec((cb, 1), lambda i: (i, 0)),
                  pl.BlockSpec((1, 1, DI), lambda i: (i, 0, 0)),
                  pl.BlockSpec((1, 1, DI), lambda i: (i, 0, 0))],
        out_specs=pl.BlockSpec((cb, DI), lambda i: (i, 0)),
        out_shape=jax.ShapeDtypeStruct((N, DI), jnp.float32),
    )(a, b, w1, w2, lng.reshape(4, 1, DI), lnb.reshape(4, 1, DI))


def _final_body(a_ref, b2_ref, w1_ref, w2_ref, res_ref, g_ref, bb_ref, o_ref):
    y = (w1_ref[...] * a_ref[...] + w2_ref[...] * b2_ref[...] + res_ref[...])
    m = jnp.mean(y, axis=-1, keepdims=True)
    v = jnp.mean((y - m) ** 2, axis=-1, keepdims=True)
    o_ref[...] = (y - m) / jnp.sqrt(v + EPS) * g_ref[...] + bb_ref[...]


def _final(a, b, w1, w2, res, g, bb):
    rb = 512
    return pl.pallas_call(
        _final_body,
        grid=(N // rb,),
        in_specs=[pl.BlockSpec((rb, D), lambda i: (i, 0)),
                  pl.BlockSpec((rb, D), lambda i: (i, 0)),
                  pl.BlockSpec((rb, 1), lambda i: (i, 0)),
                  pl.BlockSpec((rb, 1), lambda i: (i, 0)),
                  pl.BlockSpec((rb, D), lambda i: (i, 0)),
                  pl.BlockSpec((1, D), lambda i: (0, 0)),
                  pl.BlockSpec((1, D), lambda i: (0, 0))],
        out_specs=pl.BlockSpec((rb, D), lambda i: (i, 0)),
        out_shape=jax.ShapeDtypeStruct((N, D), jnp.float32),
    )(a, b, w1, w2, res, g.reshape(1, D), bb.reshape(1, D))


def _sc_gather(src, idx):
    nidx = idx.shape[0]
    dm = src.shape[1]
    rows = nidx // NWORK
    csize = 32 if dm > 768 else 64
    csize = min(csize, rows)
    mesh = plsc.VectorSubcoreMesh(core_axis_name="c", subcore_axis_name="s")

    @functools.partial(
        pl.kernel,
        out_type=jax.ShapeDtypeStruct((nidx, dm), jnp.float32),
        mesh=mesh,
        scratch_types=[pltpu.VMEM((csize,), jnp.int32),
                       pltpu.VMEM((csize, dm), jnp.float32),
                       pltpu.SemaphoreType.DMA],
    )
    def k(src_hbm, idx_hbm, out_hbm, idx_v, rows_v, sem):
        wid = lax.axis_index("s") * 2 + lax.axis_index("c")
        base = wid * rows
        for j in range(rows // csize):
            off = base + j * csize
            pltpu.sync_copy(idx_hbm.at[pl.ds(off, csize)], idx_v)
            pltpu.async_copy(src_hbm.at[idx_v], rows_v, sem).wait()
            pltpu.sync_copy(rows_v, out_hbm.at[pl.ds(off, csize)])

    return k(src, idx)


def _sc_dispatch(src, gidx, sidx):
    dm = src.shape[1]
    rows = P // NWORK
    csize = 32 if dm > 768 else 64
    mesh = plsc.VectorSubcoreMesh(core_axis_name="c", subcore_axis_name="s")

    @functools.partial(
        pl.kernel,
        out_type=jax.ShapeDtypeStruct((PPAD, dm), jnp.float32),
        mesh=mesh,
        scratch_types=[pltpu.VMEM((csize,), jnp.int32),
                       pltpu.VMEM((csize,), jnp.int32),
                       pltpu.VMEM((csize, dm), jnp.float32),
                       pltpu.SemaphoreType.DMA,
                       pltpu.SemaphoreType.DMA],
    )
    def k(src_hbm, gidx_hbm, sidx_hbm, out_hbm, gi_v, si_v, rows_v, s1, s2):
        wid = lax.axis_index("s") * 2 + lax.axis_index("c")
        base = wid * rows
        for j in range(rows // csize):
            off = base + j * csize
            pltpu.sync_copy(gidx_hbm.at[pl.ds(off, csize)], gi_v)
            pltpu.sync_copy(sidx_hbm.at[pl.ds(off, csize)], si_v)
            pltpu.async_copy(src_hbm.at[gi_v], rows_v, s1).wait()
            pltpu.async_copy(rows_v, out_hbm.at[si_v], s2).wait()

    return k(src, gidx, sidx)


def kernel(enc_input, Wq, Wk, Wv, Wfc, ln_attn_g, ln_attn_b, Wg1, W1, b1,
           W2, b2, ln1_g, ln1_b, ln2_g, ln2_b, ln3_g, ln3_b, ln4_g, ln4_b,
           lnf_g, lnf_b):
    x0 = enc_input.reshape(N, D)
    wqkv = jnp.concatenate([Wq, Wk, Wv], axis=1)
    qkv = _qkv(x0, wqkv)
    qkvh = qkv.reshape(N, 3 * H, DK).transpose(1, 0, 2)
    ctxh = _attn(qkvh)
    ctx = ctxh.transpose(1, 0, 2).reshape(N, H * DV)
    xln, e1, e2, w1, w2 = _post_attn(ctx, x0, Wfc, ln_attn_g, ln_attn_b, Wg1)

    e_pairs = jnp.concatenate([e1, e2], axis=1).reshape(P, 1)
    dst2, te2 = _route(e_pairs)
    dst = dst2.reshape(P)
    te = te2.reshape(64)[:NT]
    deven = dst[0::2]
    dodd = dst[1::2]
    gidx = (jnp.arange(P, dtype=jnp.int32) // TOPK).astype(jnp.int32)

    xs = _sc_dispatch(xln, gidx, dst)
    ys = _expert_mm(te, xs, W1, b1, D, DI)
    a1 = _sc_gather(ys, deven)
    b1g = _sc_gather(ys, dodd)
    lng = jnp.stack([ln1_g, ln2_g, ln3_g, ln4_g])
    lnb = jnp.stack([ln1_b, ln2_b, ln3_b, ln4_b])
    z = _mid(a1, b1g, w1, w2, lng, lnb)

    zs = _sc_dispatch(z, gidx, dst)
    y2s = _expert_mm(te, zs, W2, b2, DI, D)
    a2 = _sc_gather(y2s, deven)
    b2g = _sc_gather(y2s, dodd)
    out = _final(a2, b2g, w1, w2, xln, lnf_g, lnf_b)
    return out.reshape(1, N, D)
